# Initial kernel scaffold; baseline (speedup 1.0000x reference)
import jax
import jax.numpy as jnp
from jax.experimental import pallas as pl


def kernel(site, distance, connectivity, input_vol, true_vol, vol_params, energy_params):
    raise NotImplementedError("write your pallas kernel here")


# speedup vs baseline: 6.1585x; 6.1585x over previous
import functools

import jax
import jax.numpy as jnp
from jax.experimental import pallas as pl
from jax.experimental.pallas import tpu as pltpu

_MAX_ATOMIC_NUM = 84
_EMBED = 256
_NUM_MESSAGES = 6
_RBF_DIM = 128
_B, _N, _E = 8, 128, 1024
_NA = _B * _N
_NE = _B * _E
_EBLK = 512
_NBLK = _NE // _EBLK


def _gnn_body(site_ref, dist_ref, src_ref, dst_ref, dstT_ref,
              tcat_ref, bondW_ref, bondb_ref, cent_ref, gap_ref, offW_ref,
              eW1_ref, eb1_ref, eW2_ref, eb2_ref,
              nW1_ref, nb1_ref, nW2_ref, nb2_ref,
              nW3_ref, nb3_ref, nW4_ref, nb4_ref,
              out_ref, atom_s, bond_s, agg_s):
    step = pl.program_id(0)
    blk = pl.program_id(1)
    f32 = jnp.float32

    @pl.when((step == 0) & (blk == 0))
    def _init_atom():
        lane = jax.lax.broadcasted_iota(jnp.int32, (_NA, 128), 1)
        oh = (site_ref[:] == lane).astype(f32)
        cat = jnp.dot(oh, tcat_ref[:], preferred_element_type=f32)
        atom_s[:] = cat[:, :_EMBED]

    @pl.when(step == 0)
    def _init_bond():
        d = dist_ref[:]
        d = jnp.where(d != d, jnp.zeros_like(d), d)
        off = d - cent_ref[0:1, :]
        rbf = jnp.exp(-gap_ref[0:1, :] * off * off)
        b0 = jnp.dot(rbf, bondW_ref[:], preferred_element_type=f32)
        bond_s[pl.ds(blk * _EBLK, _EBLK), :] = b0 + bondb_ref[0:1, :]

    lane_a = jax.lax.broadcasted_iota(jnp.int32, (_EBLK, _NA), 1)
    oh_src = (src_ref[:] == lane_a).astype(f32)
    oh_dst = (dst_ref[:] == lane_a).astype(f32)
    atom = atom_s[:]
    src_a = jnp.dot(oh_src, atom, preferred_element_type=f32)
    dst_a = jnp.dot(oh_dst, atom, preferred_element_type=f32)

    bond = bond_s[pl.ds(blk * _EBLK, _EBLK), :]
    eb1 = eb1_ref[0]
    h = (jnp.dot(bond, eW1_ref[0, :_EMBED, :], preferred_element_type=f32)
         + jnp.dot(src_a, eW1_ref[0, _EMBED:2 * _EMBED, :], preferred_element_type=f32)
         + jnp.dot(dst_a, eW1_ref[0, 2 * _EMBED:, :], preferred_element_type=f32)
         + eb1)
    h = jnp.maximum(h, 0.0)
    h = jnp.dot(h, eW2_ref[0], preferred_element_type=f32) + eb2_ref[0]
    bond_new = bond + h
    bond_s[pl.ds(blk * _EBLK, _EBLK), :] = bond_new

    m = (jnp.dot(src_a, nW1_ref[0, :_EMBED, :], preferred_element_type=f32)
         + jnp.dot(bond_new, nW1_ref[0, _EMBED:, :], preferred_element_type=f32)
         + nb1_ref[0])
    m = jnp.maximum(m, 0.0)
    m = jnp.dot(m, nW2_ref[0], preferred_element_type=f32) + nb2_ref[0]
    dmask = (dist_ref[:] != 0.0).astype(f32)
    m = m * dmask

    row_i = jax.lax.broadcasted_iota(jnp.int32, (_NA, _EBLK), 0)
    ohT = (row_i == dstT_ref[0]).astype(f32)
    contrib = jnp.dot(ohT, m, preferred_element_type=f32)

    @pl.when(blk == 0)
    def _agg_set():
        agg_s[:] = contrib

    @pl.when(blk != 0)
    def _agg_add():
        agg_s[:] = agg_s[:] + contrib

    @pl.when(blk == _NBLK - 1)
    def _node_update():
        a = jnp.dot(agg_s[:], nW3_ref[0], preferred_element_type=f32) + nb3_ref[0]
        a = jnp.maximum(a, 0.0)
        a = jnp.dot(a, nW4_ref[0], preferred_element_type=f32) + nb4_ref[0]
        atom_s[:] = atom_s[:] + a

    @pl.when((step == _NUM_MESSAGES - 1) & (blk == _NBLK - 1))
    def _readout():
        lane = jax.lax.broadcasted_iota(jnp.int32, (_NA, 128), 1)
        oh = (site_ref[:] == lane).astype(f32)
        cat = jnp.dot(oh, tcat_ref[:], preferred_element_type=f32)
        meanv = cat[:, _EMBED:_EMBED + 1]
        offv = jnp.dot(atom_s[:], offW_ref[:], preferred_element_type=f32)
        val = meanv + offv[:, 0:1]
        maskf = (site_ref[:] != 0).astype(f32)
        col = jax.lax.broadcasted_iota(jnp.int32, (_B, _NA), 1)
        row = jax.lax.broadcasted_iota(jnp.int32, (_B, _NA), 0)
        boh = (col // _N == row).astype(f32)
        sums = jnp.dot(boh, val * maskf, preferred_element_type=f32)
        cnts = jnp.dot(boh, maskf, preferred_element_type=f32)
        out_ref[:] = sums / jnp.maximum(cnts, 1.0)


@functools.partial(jax.jit, static_argnames=())
def _gnn_pass(site_flat, dist_flat, src_g, dst_g, dstT, tcat, bondW, bondb,
              cent, gap, offW, ew):
    grid = (_NUM_MESSAGES, _NBLK)

    def c3(i):
        return pl.BlockSpec((1,) + i, lambda s, b: (s, 0, 0))

    const2 = lambda shp: pl.BlockSpec(shp, lambda s, b: (0, 0))
    eblk2 = pl.BlockSpec((_EBLK, 1), lambda s, b: (b, 0))

    in_specs = [
        const2((_NA, 1)),
        eblk2,
        eblk2,
        eblk2,
        pl.BlockSpec((1, 1, _EBLK), lambda s, b: (b, 0, 0)),
        const2((128, 384)),
        const2((_RBF_DIM, _EMBED)),
        const2((8, _EMBED)),
        const2((8, 128)),
        const2((8, 128)),
        const2((_EMBED, 128)),
        c3((3 * _EMBED, 2 * _EMBED)),
        pl.BlockSpec((1, 1, 2 * _EMBED), lambda s, b: (s, 0, 0)),
        c3((2 * _EMBED, _EMBED)),
        pl.BlockSpec((1, 1, _EMBED), lambda s, b: (s, 0, 0)),
        c3((2 * _EMBED, 2 * _EMBED)),
        pl.BlockSpec((1, 1, 2 * _EMBED), lambda s, b: (s, 0, 0)),
        c3((2 * _EMBED, _EMBED)),
        pl.BlockSpec((1, 1, _EMBED), lambda s, b: (s, 0, 0)),
        c3((_EMBED, 2 * _EMBED)),
        pl.BlockSpec((1, 1, 2 * _EMBED), lambda s, b: (s, 0, 0)),
        c3((2 * _EMBED, _EMBED)),
        pl.BlockSpec((1, 1, _EMBED), lambda s, b: (s, 0, 0)),
    ]

    return pl.pallas_call(
        _gnn_body,
        grid=grid,
        in_specs=in_specs,
        out_specs=pl.BlockSpec((_B, 1), lambda s, b: (0, 0)),
        out_shape=jax.ShapeDtypeStruct((_B, 1), jnp.float32),
        scratch_shapes=[
            pltpu.VMEM((_NA, _EMBED), jnp.float32),
            pltpu.VMEM((_NE, _EMBED), jnp.float32),
            pltpu.VMEM((_NA, _EMBED), jnp.float32),
        ],
        compiler_params=pltpu.CompilerParams(
            dimension_semantics=("arbitrary", "arbitrary"),
        ),
    )(site_flat, dist_flat, src_g, dst_g, dstT, tcat, bondW, bondb,
      cent, gap, offW, *ew)


def _prep_params(p):
    emb = jnp.zeros((128, 384), jnp.float32)
    emb = emb.at[:_MAX_ATOMIC_NUM, :_EMBED].set(p['atom_embedding'])
    emb = emb.at[:_MAX_ATOMIC_NUM, _EMBED:_EMBED + 1].set(p['atom_mean'])
    emb = emb.at[:, _EMBED].add(p['off_b'][0])
    cent = jnp.broadcast_to(p['rbf_centers'][None, :], (8, _RBF_DIM))
    gap = jnp.full((8, 128), p['rbf_gap'], jnp.float32)
    bondb = jnp.broadcast_to(p['bond_b'][None, :], (8, _EMBED))
    offW = jnp.zeros((_EMBED, 128), jnp.float32)
    offW = offW.at[:, 0:1].set(p['off_W'])

    def stk(group, name):
        return jnp.stack([p[group][i][name] for i in range(_NUM_MESSAGES)])

    def stkb(group, name):
        return jnp.stack([p[group][i][name] for i in range(_NUM_MESSAGES)])[:, None, :]

    ew = (stk('edge', 'W1'), stkb('edge', 'b1'), stk('edge', 'W2'), stkb('edge', 'b2'),
          stk('node', 'W1'), stkb('node', 'b1'), stk('node', 'W2'), stkb('node', 'b2'),
          stk('node', 'W3'), stkb('node', 'b3'), stk('node', 'W4'), stkb('node', 'b4'))
    return emb, cent, gap, p['bond_W'], bondb, offW, ew


def kernel(site, distance, connectivity, input_vol, true_vol, vol_params,
           energy_params):
    site = site.astype(jnp.int32)
    conn = connectivity.astype(jnp.int32)
    site_flat = site.reshape(_NA, 1)
    offs = (jnp.arange(_B, dtype=jnp.int32) * _N)[:, None]
    dst_g = (conn[:, :, 0] + offs).reshape(_NE, 1)
    src_g = (conn[:, :, 1] + offs).reshape(_NE, 1)
    dstT = dst_g.reshape(_NBLK, 1, _EBLK)

    outs = []
    dist = distance
    for params in (vol_params, energy_params):
        emb, cent, gap, bondW, bondb, offW, ew = _prep_params(params)
        dist_flat = dist.reshape(_NE, 1)
        pooled = _gnn_pass(site_flat, dist_flat, src_g, dst_g, dstT, emb,
                           bondW, bondb, cent, gap, offW, ew)
        outs.append(pooled)
        if params is vol_params:
            dist = distance * jnp.power(outs[0] / input_vol, 1.0 / 3.0)
    return outs[0], outs[1]
